# Initial kernel scaffold; baseline (speedup 1.0000x reference)
import jax
import jax.numpy as jnp
from jax.experimental import pallas as pl


def kernel(H, Z, block_id, batch_id, edges, edge_attr, params):
    raise NotImplementedError("write your pallas kernel here")


# speedup vs baseline: 1.4157x; 1.4157x over previous
import functools

import numpy as np
import jax
import jax.numpy as jnp
from jax import lax
from jax.experimental import pallas as pl
from jax.experimental.pallas import tpu as pltpu
from jax.experimental.pallas import tpu_sc as plsc

N_ATOMS = 50000
N_BLOCKS = 10000
N_GRAPHS = 64
N_EDGES = 160000
HIDDEN = 256
EDGE_SIZE = 16
NUM_GAUSSIANS = 50
N_LAYERS = 3
CUTOFF = 10.0
LOG2 = float(np.log(2.0))

NC = 2
NS = 16
LANES = 16
HALF = HIDDEN // 2
CHUNK = 128
A_PAD = 51200
E_PAD = 163840
ROWS_B = N_BLOCKS // NS
NT = 1000
TE = 512

@functools.cache
def _mesh():
    return plsc.VectorSubcoreMesh(
        core_axis_name="c", subcore_axis_name="s", num_cores=NC, num_subcores=NS)


def _zero_shared_slice(buf, width_groups, shared, sid):
    def zl(i, _):
        for j in range(width_groups):
            buf[i, pl.ds(j * LANES, LANES)] = jnp.zeros((LANES,), jnp.float32)
        return 0
    lax.fori_loop(0, CHUNK, zl, 0)
    piece = ROWS_B // 5
    for k in range(5):
        pltpu.sync_copy(buf.at[pl.ds(0, piece)],
                        shared.at[pl.ds(sid * ROWS_B + k * piece, piece)])


def _sc_atoms_body(hs_ref, zp_ref, bid_ref, hbsum_ref, zc_ref,
                   hsh, zsh, vh, vz, idxv, sem):
    del sem
    cid = lax.axis_index("c")
    sid = lax.axis_index("s")
    _zero_shared_slice(vh, HALF // LANES, hsh, sid)
    _zero_shared_slice(vz, 1, zsh, sid)
    plsc.subcore_barrier()
    tile_rows = A_PAD // NS
    tile_base = sid * tile_rows

    def chunk(k, _):
        base = tile_base + k * CHUNK
        pltpu.sync_copy(bid_ref.at[pl.ds(base, CHUNK)], idxv)
        pltpu.sync_copy(hs_ref.at[pl.ds(cid * A_PAD + base, CHUNK)], vh)
        pltpu.sync_copy(zp_ref.at[pl.ds(base, CHUNK)], vz)
        pltpu.sync_copy(vh, hsh.at[idxv], add=True)
        pltpu.sync_copy(vz, zsh.at[idxv], add=True)
        return 0

    lax.fori_loop(0, tile_rows // CHUNK, chunk, 0)
    plsc.subcore_barrier()
    r0 = sid * ROWS_B
    out0 = cid * N_BLOCKS + r0
    pltpu.sync_copy(hsh.at[pl.ds(r0, ROWS_B)], hbsum_ref.at[pl.ds(out0, ROWS_B)])
    pltpu.sync_copy(zsh.at[pl.ds(r0, ROWS_B)], zc_ref.at[pl.ds(out0, ROWS_B)])


def _sc_atoms_call(hsplit, zp, bid):
    return pl.kernel(
        _sc_atoms_body,
        out_type=(jax.ShapeDtypeStruct((NC * N_BLOCKS, HALF), jnp.float32),
                  jax.ShapeDtypeStruct((NC * N_BLOCKS, LANES), jnp.float32)),
        mesh=_mesh(),
        compiler_params=pltpu.CompilerParams(use_tc_tiling_on_sc=False),
        scratch_types=[
            pltpu.VMEM_SHARED((N_BLOCKS, HALF), jnp.float32),
            pltpu.VMEM_SHARED((N_BLOCKS, LANES), jnp.float32),
            pltpu.VMEM((CHUNK, HALF), jnp.float32),
            pltpu.VMEM((CHUNK, LANES), jnp.float32),
            pltpu.VMEM((CHUNK,), jnp.int32),
            pltpu.SemaphoreType.DMA,
        ],
    )(hsplit, zp, bid)


def _sc_dvec_body(zb_ref, row_ref, col_ref, dvec_ref,
                  zr, zc, db, rv, cv, sem):
    cid = lax.axis_index("c")
    sid = lax.axis_index("s")
    w = sid * NC + cid
    per_w = E_PAD // (NC * NS)

    def chunk(k, _):
        base = w * per_w + k * CHUNK
        pltpu.sync_copy(row_ref.at[pl.ds(base, CHUNK)], rv)
        pltpu.sync_copy(col_ref.at[pl.ds(base, CHUNK)], cv)
        pltpu.async_copy(zb_ref.at[rv], zr, sem).wait()
        pltpu.async_copy(zb_ref.at[cv], zc, sem).wait()

        def sub(i, _):
            s = pl.ds(0, LANES)
            db[i, s] = zr[i, s] - zc[i, s]
            return 0

        lax.fori_loop(0, CHUNK, sub, 0)
        pltpu.sync_copy(db, dvec_ref.at[pl.ds(base, CHUNK)])
        return 0

    lax.fori_loop(0, per_w // CHUNK, chunk, 0)


def _sc_dvec_call(zb16, rowp, colp):
    return pl.kernel(
        _sc_dvec_body,
        out_type=jax.ShapeDtypeStruct((E_PAD, LANES), jnp.float32),
        mesh=_mesh(),
        compiler_params=pltpu.CompilerParams(use_tc_tiling_on_sc=False),
        scratch_types=[
            pltpu.VMEM((CHUNK, LANES), jnp.float32),
            pltpu.VMEM((CHUNK, LANES), jnp.float32),
            pltpu.VMEM((CHUNK, LANES), jnp.float32),
            pltpu.VMEM((CHUNK,), jnp.int32),
            pltpu.VMEM((CHUNK,), jnp.int32),
            pltpu.SemaphoreType.DMA,
        ],
    )(zb16, rowp, colp)


def _sc_edge_body(hw_ref, wf_ref, row_ref, col_ref, agg_ref,
                  ash, xb, wb, rv, cv, cv2, sem):
    cid = lax.axis_index("c")
    sid = lax.axis_index("s")
    _zero_shared_slice(xb, HALF // LANES, ash, sid)
    plsc.subcore_barrier()
    tile_edges = E_PAD // NS
    off = cid * N_BLOCKS

    def chunk(k, _):
        base = sid * tile_edges + k * CHUNK
        pltpu.sync_copy(row_ref.at[pl.ds(base, CHUNK)], rv)
        pltpu.sync_copy(col_ref.at[pl.ds(base, CHUNK)], cv)

        def al(j, _):
            s = pl.ds(j * LANES, LANES)
            cv2[s] = cv[s] + off
            return 0

        lax.fori_loop(0, CHUNK // LANES, al, 0)
        pltpu.async_copy(hw_ref.at[cv2], xb, sem).wait()
        pltpu.sync_copy(wf_ref.at[pl.ds(cid * E_PAD + base, CHUNK)], wb)

        def ml(i, _):
            for j in range(HALF // LANES):
                s = pl.ds(j * LANES, LANES)
                xb[i, s] = xb[i, s] * wb[i, s]
            return 0

        lax.fori_loop(0, CHUNK, ml, 0)
        pltpu.sync_copy(xb, ash.at[rv], add=True)
        return 0

    lax.fori_loop(0, tile_edges // CHUNK, chunk, 0)
    plsc.subcore_barrier()
    r0 = sid * ROWS_B
    pltpu.sync_copy(ash.at[pl.ds(r0, ROWS_B)],
                    agg_ref.at[pl.ds(cid * N_BLOCKS + r0, ROWS_B)])


def _sc_edge_call(hw2, wf2, rowp, colp):
    return pl.kernel(
        _sc_edge_body,
        out_type=jax.ShapeDtypeStruct((NC * N_BLOCKS, HALF), jnp.float32),
        mesh=_mesh(),
        compiler_params=pltpu.CompilerParams(use_tc_tiling_on_sc=False),
        scratch_types=[
            pltpu.VMEM_SHARED((N_BLOCKS, HALF), jnp.float32),
            pltpu.VMEM((CHUNK, HALF), jnp.float32),
            pltpu.VMEM((CHUNK, HALF), jnp.float32),
            pltpu.VMEM((CHUNK,), jnp.int32),
            pltpu.VMEM((CHUNK,), jnp.int32),
            pltpu.VMEM((CHUNK,), jnp.int32),
            pltpu.SemaphoreType.DMA,
        ],
    )(hw2, wf2, rowp, colp)


def _split_body(h_ref, o_ref):
    o_ref[0] = h_ref[:, :HALF]
    o_ref[1] = h_ref[:, HALF:]


def _tc_split(h_pad):
    return pl.pallas_call(
        _split_body,
        grid=(A_PAD // TE,),
        in_specs=[pl.BlockSpec((TE, HIDDEN), lambda t: (t, 0))],
        out_specs=pl.BlockSpec((2, TE, HALF), lambda t: (0, t, 0)),
        out_shape=jax.ShapeDtypeStruct((2, A_PAD, HALF), jnp.float32),
    )(h_pad)


def _mean_body(hb_ref, zc_ref, hbo_ref, zbo_ref):
    zc = zc_ref[...]
    lane = lax.broadcasted_iota(jnp.int32, zc.shape, 1)
    cnt = jnp.sum(jnp.where(lane == 3, zc, 0.0), axis=1, keepdims=True)
    inv = 1.0 / jnp.maximum(cnt, 1.0)
    hbo_ref[:, :HALF] = hb_ref[0] * inv
    hbo_ref[:, HALF:] = hb_ref[1] * inv
    zbo_ref[...] = jnp.where(lane == 3, 0.0, zc * inv)


def _tc_mean(hbsum, zc):
    return pl.pallas_call(
        _mean_body,
        grid=(N_BLOCKS // NT,),
        in_specs=[pl.BlockSpec((2, NT, HALF), lambda t: (0, t, 0)),
                  pl.BlockSpec((NT, LANES), lambda t: (t, 0))],
        out_specs=[pl.BlockSpec((NT, HIDDEN), lambda t: (t, 0)),
                   pl.BlockSpec((NT, LANES), lambda t: (t, 0))],
        out_shape=[jax.ShapeDtypeStruct((N_BLOCKS, HIDDEN), jnp.float32),
                   jax.ShapeDtypeStruct((N_BLOCKS, LANES), jnp.float32)],
    )(hbsum, zc)


def _wf_body(dv_ref, ea_ref, ew_ref, eb_ref, w1_ref, b1_ref, w2_ref, b2_ref,
             o0, o1, o2):
    t = pl.program_id(0)
    dv = dv_ref[...]
    dist = jnp.sqrt(jnp.sum(dv * dv, axis=1, keepdims=True) + 1e-12)
    spacing = CUTOFF / (NUM_GAUSSIANS - 1)
    coeff = -0.5 / spacing ** 2
    offs = lax.broadcasted_iota(
        jnp.int32, (TE, NUM_GAUSSIANS), 1).astype(jnp.float32) * spacing
    ef = jnp.exp(coeff * (dist - offs) ** 2)
    eap = jnp.dot(ea_ref[...], ew_ref[...],
                  preferred_element_type=jnp.float32) + eb_ref[...]
    ef = ef + eap
    cutoff = 0.5 * (jnp.cos(dist * (np.pi / CUTOFF)) + 1.0)
    cutoff = cutoff * (dist < CUTOFF).astype(jnp.float32)
    rows = lax.broadcasted_iota(jnp.int32, (TE, 1), 0) + t * TE
    scale = cutoff * (rows < N_EDGES).astype(jnp.float32)
    for l, o in enumerate((o0, o1, o2)):
        s = jnp.dot(ef, w1_ref[l], preferred_element_type=jnp.float32) + b1_ref[l]
        s = jax.nn.softplus(s) - LOG2
        wf = jnp.dot(s, w2_ref[l], preferred_element_type=jnp.float32) + b2_ref[l]
        wf = wf * scale
        o[0] = wf[:, :HALF]
        o[1] = wf[:, HALF:]


def _tc_wf(dvec, ea, ew, eb, w1s, b1s, w2s, b2s):
    full = lambda *shape: pl.BlockSpec(shape, lambda t: (0,) * len(shape))
    wf_spec = pl.BlockSpec((2, TE, HALF), lambda t: (0, t, 0))
    wf_shape = jax.ShapeDtypeStruct((2, E_PAD, HALF), jnp.float32)
    return pl.pallas_call(
        _wf_body,
        grid=(E_PAD // TE,),
        in_specs=[pl.BlockSpec((TE, LANES), lambda t: (t, 0)),
                  pl.BlockSpec((TE, EDGE_SIZE), lambda t: (t, 0)),
                  full(EDGE_SIZE, NUM_GAUSSIANS),
                  full(1, NUM_GAUSSIANS),
                  full(N_LAYERS, NUM_GAUSSIANS, HIDDEN),
                  full(N_LAYERS, HIDDEN),
                  full(N_LAYERS, HIDDEN, HIDDEN),
                  full(N_LAYERS, HIDDEN)],
        out_specs=[wf_spec, wf_spec, wf_spec],
        out_shape=[wf_shape, wf_shape, wf_shape],
    )(dvec, ea, ew, eb, w1s, b1s, w2s, b2s)


def _hw_body(h_ref, w_ref, o_ref):
    r = jnp.dot(h_ref[...], w_ref[...], preferred_element_type=jnp.float32)
    o_ref[0] = r[:, :HALF]
    o_ref[1] = r[:, HALF:]


def _tc_hw(h, w):
    return pl.pallas_call(
        _hw_body,
        grid=(N_BLOCKS // NT,),
        in_specs=[pl.BlockSpec((NT, HIDDEN), lambda t: (t, 0)),
                  pl.BlockSpec((HIDDEN, HIDDEN), lambda t: (0, 0))],
        out_specs=pl.BlockSpec((2, NT, HALF), lambda t: (0, t, 0)),
        out_shape=jax.ShapeDtypeStruct((2, N_BLOCKS, HALF), jnp.float32),
    )(h, w)


def _post_body(a_ref, h_ref, w2_ref, b2_ref, ow_ref, ob_ref, o_ref):
    x = (jnp.dot(a_ref[0], w2_ref[:HALF, :], preferred_element_type=jnp.float32)
         + jnp.dot(a_ref[1], w2_ref[HALF:, :], preferred_element_type=jnp.float32)
         + b2_ref[...])
    x = jax.nn.softplus(x) - LOG2
    x = jnp.dot(x, ow_ref[...], preferred_element_type=jnp.float32) + ob_ref[...]
    o_ref[...] = h_ref[...] + x


def _tc_post(agg, h, w2, b2, ow, ob):
    return pl.pallas_call(
        _post_body,
        grid=(N_BLOCKS // NT,),
        in_specs=[pl.BlockSpec((2, NT, HALF), lambda t: (0, t, 0)),
                  pl.BlockSpec((NT, HIDDEN), lambda t: (t, 0)),
                  pl.BlockSpec((HIDDEN, HIDDEN), lambda t: (0, 0)),
                  pl.BlockSpec((1, HIDDEN), lambda t: (0, 0)),
                  pl.BlockSpec((HIDDEN, HIDDEN), lambda t: (0, 0)),
                  pl.BlockSpec((1, HIDDEN), lambda t: (0, 0))],
        out_specs=pl.BlockSpec((NT, HIDDEN), lambda t: (t, 0)),
        out_shape=jax.ShapeDtypeStruct((N_BLOCKS, HIDDEN), jnp.float32),
    )(agg, h, w2, b2, ow, ob)


def _norm_body(x_ref, o_ref):
    x = x_ref[...]
    n = jnp.sqrt(jnp.sum(x * x, axis=1, keepdims=True))
    o_ref[...] = x / jnp.maximum(n, 1e-12)


def _tc_norm(x, tile):
    n = x.shape[0]
    return pl.pallas_call(
        _norm_body,
        grid=(n // tile,),
        in_specs=[pl.BlockSpec((tile, HIDDEN), lambda t: (t, 0))],
        out_specs=pl.BlockSpec((tile, HIDDEN), lambda t: (t, 0)),
        out_shape=jax.ShapeDtypeStruct((n, HIDDEN), jnp.float32),
    )(x)


def _pool_body(bid_ref, br_ref, o_ref):
    t = pl.program_id(0)

    @pl.when(t == 0)
    def _():
        o_ref[...] = jnp.zeros_like(o_ref)

    bid = bid_ref[0, 0, :]
    gi = lax.broadcasted_iota(jnp.int32, (N_GRAPHS, NT), 0)
    oh = (gi == bid[None, :]).astype(jnp.float32)
    o_ref[...] += jnp.dot(oh, br_ref[...], preferred_element_type=jnp.float32)


def _tc_pool(bid3, br):
    return pl.pallas_call(
        _pool_body,
        grid=(N_BLOCKS // NT,),
        in_specs=[pl.BlockSpec((1, 1, NT), lambda t: (t, 0, 0)),
                  pl.BlockSpec((NT, HIDDEN), lambda t: (t, 0))],
        out_specs=pl.BlockSpec((N_GRAPHS, HIDDEN), lambda t: (0, 0)),
        out_shape=jax.ShapeDtypeStruct((N_GRAPHS, HIDDEN), jnp.float32),
    )(bid3, br)


def kernel(H, Z, block_id, batch_id, edges, edge_attr, params):
    f32 = jnp.float32
    H = H.astype(f32)
    Z = Z.astype(f32)
    edge_attr = edge_attr.astype(f32)
    bid_p = jnp.pad(block_id.astype(jnp.int32), (0, A_PAD - N_ATOMS))
    rowp = jnp.pad(edges[0].astype(jnp.int32), (0, E_PAD - N_EDGES))
    colp = jnp.pad(edges[1].astype(jnp.int32), (0, E_PAD - N_EDGES))

    h_pad = jnp.pad(H, ((0, A_PAD - N_ATOMS), (0, 0)))
    zlane = jnp.concatenate(
        [Z, jnp.ones((N_ATOMS, 1), f32), jnp.zeros((N_ATOMS, 12), f32)], axis=1)
    zp = jnp.pad(zlane, ((0, A_PAD - N_ATOMS), (0, 0)))
    ea_p = jnp.pad(edge_attr, ((0, E_PAD - N_EDGES), (0, 0)))

    hsplit = _tc_split(h_pad).reshape(2 * A_PAD, HALF)
    hbsum2, zc2 = _sc_atoms_call(hsplit, zp, bid_p)
    hb, zb16 = _tc_mean(hbsum2.reshape(2, N_BLOCKS, HALF), zc2[:N_BLOCKS])

    dvec = _sc_dvec_call(zb16, rowp, colp)

    lp = params['layers']
    w1s = jnp.stack([l['mlp_w1'] for l in lp]).astype(f32)
    b1s = jnp.stack([l['mlp_b1'] for l in lp]).astype(f32)
    w2s = jnp.stack([l['mlp_w2'] for l in lp]).astype(f32)
    b2s = jnp.stack([l['mlp_b2'] for l in lp]).astype(f32)
    wfs = _tc_wf(dvec, ea_p,
                 params['edge_linear_w'].astype(f32),
                 params['edge_linear_b'].astype(f32).reshape(1, NUM_GAUSSIANS),
                 w1s, b1s, w2s, b2s)

    h = hb
    for l in range(N_LAYERS):
        hw2 = _tc_hw(h, lp[l]['conv_lin1_w'].astype(f32))
        agg2 = _sc_edge_call(hw2.reshape(2 * N_BLOCKS, HALF),
                             wfs[l].reshape(2 * E_PAD, HALF), rowp, colp)
        h = _tc_post(agg2.reshape(2, N_BLOCKS, HALF), h,
                     lp[l]['conv_lin2_w'].astype(f32),
                     lp[l]['conv_lin2_b'].astype(f32).reshape(1, HIDDEN),
                     lp[l]['out_w'].astype(f32),
                     lp[l]['out_b'].astype(f32).reshape(1, HIDDEN))

    block_repr = _tc_norm(h, NT)
    graph = _tc_pool(batch_id.astype(jnp.int32).reshape(N_BLOCKS // NT, 1, NT),
                     block_repr)
    graph_repr = _tc_norm(graph, N_GRAPHS)
    return (hb, block_repr, graph_repr, None)


# speedup vs baseline: 1.7277x; 1.2204x over previous
import functools

import numpy as np
import jax
import jax.numpy as jnp
from jax import lax
from jax.experimental import pallas as pl
from jax.experimental.pallas import tpu as pltpu
from jax.experimental.pallas import tpu_sc as plsc

N_ATOMS = 50000
N_BLOCKS = 10000
N_GRAPHS = 64
N_EDGES = 160000
HIDDEN = 256
EDGE_SIZE = 16
NUM_GAUSSIANS = 50
N_LAYERS = 3
CUTOFF = 10.0
LOG2 = float(np.log(2.0))

NC = 2
NS = 16
LANES = 16
HALF = HIDDEN // 2
CHUNK = 128
A_PAD = 51200
E_PAD = 163840
ROWS_B = N_BLOCKS // NS
NT = 1000
TE = 512

@functools.cache
def _mesh():
    return plsc.VectorSubcoreMesh(
        core_axis_name="c", subcore_axis_name="s", num_cores=NC, num_subcores=NS)


def _zero_shared_slice(buf, rows, width_groups, shared, sid):
    def zl(i, _):
        for j in range(width_groups):
            buf[i, pl.ds(j * LANES, LANES)] = jnp.zeros((LANES,), jnp.float32)
        return 0
    lax.fori_loop(0, rows, zl, 0)
    full, rem = ROWS_B // rows, ROWS_B % rows
    for k in range(full):
        pltpu.sync_copy(buf.at[pl.ds(0, rows)],
                        shared.at[pl.ds(sid * ROWS_B + k * rows, rows)])
    if rem:
        pltpu.sync_copy(buf.at[pl.ds(0, rem)],
                        shared.at[pl.ds(sid * ROWS_B + full * rows, rem)])


def _sc_atoms_body(hs_ref, zp_ref, bid_ref, hbsum_ref, zc_ref,
                   hsh, zsh, vh, vz, idxv, sem):
    del sem
    cid = lax.axis_index("c")
    sid = lax.axis_index("s")
    _zero_shared_slice(vh, CHUNK, HALF // LANES, hsh, sid)
    _zero_shared_slice(vz, CHUNK, 1, zsh, sid)
    plsc.subcore_barrier()
    tile_rows = A_PAD // NS
    tile_base = sid * tile_rows

    def chunk(k, _):
        base = tile_base + k * CHUNK
        pltpu.sync_copy(bid_ref.at[pl.ds(base, CHUNK)], idxv)
        pltpu.sync_copy(hs_ref.at[pl.ds(cid * A_PAD + base, CHUNK)], vh)
        pltpu.sync_copy(zp_ref.at[pl.ds(base, CHUNK)], vz)
        pltpu.sync_copy(vh, hsh.at[idxv], add=True)
        pltpu.sync_copy(vz, zsh.at[idxv], add=True)
        return 0

    lax.fori_loop(0, tile_rows // CHUNK, chunk, 0)
    plsc.subcore_barrier()
    r0 = sid * ROWS_B
    out0 = cid * N_BLOCKS + r0
    pltpu.sync_copy(hsh.at[pl.ds(r0, ROWS_B)], hbsum_ref.at[pl.ds(out0, ROWS_B)])
    pltpu.sync_copy(zsh.at[pl.ds(r0, ROWS_B)], zc_ref.at[pl.ds(out0, ROWS_B)])


def _sc_atoms_call(hsplit, zp, bid):
    return pl.kernel(
        _sc_atoms_body,
        out_type=(jax.ShapeDtypeStruct((NC * N_BLOCKS, HALF), jnp.float32),
                  jax.ShapeDtypeStruct((NC * N_BLOCKS, LANES), jnp.float32)),
        mesh=_mesh(),
        compiler_params=pltpu.CompilerParams(use_tc_tiling_on_sc=False),
        scratch_types=[
            pltpu.VMEM_SHARED((N_BLOCKS, HALF), jnp.float32),
            pltpu.VMEM_SHARED((N_BLOCKS, LANES), jnp.float32),
            pltpu.VMEM((CHUNK, HALF), jnp.float32),
            pltpu.VMEM((CHUNK, LANES), jnp.float32),
            pltpu.VMEM((CHUNK,), jnp.int32),
            pltpu.SemaphoreType.DMA,
        ],
    )(hsplit, zp, bid)


def _sc_dvec_body(zb_ref, row_ref, col_ref, dvec_ref,
                  zr, zc, db, rv, cv, sem):
    cid = lax.axis_index("c")
    sid = lax.axis_index("s")
    w = sid * NC + cid
    per_w = E_PAD // (NC * NS)

    def chunk(k, _):
        base = w * per_w + k * CHUNK
        pltpu.sync_copy(row_ref.at[pl.ds(base, CHUNK)], rv)
        pltpu.sync_copy(col_ref.at[pl.ds(base, CHUNK)], cv)
        pltpu.async_copy(zb_ref.at[rv], zr, sem).wait()
        pltpu.async_copy(zb_ref.at[cv], zc, sem).wait()

        def sub(i, _):
            s = pl.ds(0, LANES)
            db[i, s] = zr[i, s] - zc[i, s]
            return 0

        lax.fori_loop(0, CHUNK, sub, 0)
        pltpu.sync_copy(db, dvec_ref.at[pl.ds(base, CHUNK)])
        return 0

    lax.fori_loop(0, per_w // CHUNK, chunk, 0)


def _sc_dvec_call(zb16, rowp, colp):
    return pl.kernel(
        _sc_dvec_body,
        out_type=jax.ShapeDtypeStruct((E_PAD, LANES), jnp.float32),
        mesh=_mesh(),
        compiler_params=pltpu.CompilerParams(use_tc_tiling_on_sc=False),
        scratch_types=[
            pltpu.VMEM((CHUNK, LANES), jnp.float32),
            pltpu.VMEM((CHUNK, LANES), jnp.float32),
            pltpu.VMEM((CHUNK, LANES), jnp.float32),
            pltpu.VMEM((CHUNK,), jnp.int32),
            pltpu.VMEM((CHUNK,), jnp.int32),
            pltpu.SemaphoreType.DMA,
        ],
    )(zb16, rowp, colp)


C64 = 64
BCH = 80
NBATCH = E_PAD // NS // C64 // BCH


def _sc_edge_body(hw_ref, wf_ref, row2_ref, col2_ref, agg_ref,
                  ash, xb0, xb1, wb0, wb1, rvb, cvb,
                  sg0, sg1, sw0, sw1, ss0, ss1):
    cid = lax.axis_index("c")
    sid = lax.axis_index("s")
    _zero_shared_slice(xb0, C64, HALF // LANES, ash, sid)
    plsc.subcore_barrier()
    off = cid * N_BLOCKS
    wf_row0 = cid * (E_PAD // C64)
    xb = (xb0, xb1)
    wb = (wb0, wb1)
    sg = (sg0, sg1)
    sw = (sw0, sw1)
    ss = (ss0, ss1)

    def mul(b):
        def ml(i, _):
            for q in range(HALF // LANES):
                s = pl.ds(q * LANES, LANES)
                xb[b][i, s] = xb[b][i, s] * wb[b][i, s]
            return 0
        lax.fori_loop(0, C64, ml, 0)

    for bb in range(NBATCH):
        g0 = sid * (NBATCH * BCH) + bb * BCH
        pltpu.sync_copy(row2_ref.at[pl.ds(g0, BCH)], rvb)
        pltpu.sync_copy(col2_ref.at[pl.ds(g0, BCH)], cvb)

        def addoff(r, _):
            for q in range(C64 // LANES):
                s = pl.ds(q * LANES, LANES)
                cvb[r, s] = cvb[r, s] + off
            return 0

        lax.fori_loop(0, BCH, addoff, 0)

        def gather(j, b):
            pltpu.async_copy(hw_ref.at[cvb.at[j]], xb[b], sg[b])
            pltpu.async_copy(wf_ref.at[pl.ds((wf_row0 + g0 + j) * C64, C64)],
                             wb[b], sw[b])

        def wait_gw(j, b):
            pltpu.make_async_copy(hw_ref.at[cvb.at[j]], xb[b], sg[b]).wait()
            pltpu.make_async_copy(
                wf_ref.at[pl.ds((wf_row0 + g0 + j) * C64, C64)],
                wb[b], sw[b]).wait()

        def scat(j, b):
            pltpu.async_copy(xb[b], ash.at[rvb.at[j]], ss[b], add=True)

        def wait_s(j, b):
            pltpu.make_async_copy(xb[b], ash.at[rvb.at[j]], ss[b]).wait()

        gather(0, 0)
        gather(1, 1)

        def step(kk, _):
            j0 = 2 * kk
            wait_gw(j0, 0)
            mul(0)
            scat(j0, 0)
            wait_gw(j0 + 1, 1)
            mul(1)
            scat(j0 + 1, 1)
            wait_s(j0, 0)
            gather(j0 + 2, 0)
            wait_s(j0 + 1, 1)
            gather(j0 + 3, 1)
            return 0

        lax.fori_loop(0, BCH // 2 - 1, step, 0)
        wait_gw(BCH - 2, 0)
        mul(0)
        scat(BCH - 2, 0)
        wait_gw(BCH - 1, 1)
        mul(1)
        scat(BCH - 1, 1)
        wait_s(BCH - 2, 0)
        wait_s(BCH - 1, 1)

    plsc.subcore_barrier()
    r0 = sid * ROWS_B
    pltpu.sync_copy(ash.at[pl.ds(r0, ROWS_B)],
                    agg_ref.at[pl.ds(cid * N_BLOCKS + r0, ROWS_B)])


def _sc_edge_call(hw2, wf2, row2, col2):
    return pl.kernel(
        _sc_edge_body,
        out_type=jax.ShapeDtypeStruct((NC * N_BLOCKS, HALF), jnp.float32),
        mesh=_mesh(),
        compiler_params=pltpu.CompilerParams(use_tc_tiling_on_sc=False),
        scratch_types=[
            pltpu.VMEM_SHARED((N_BLOCKS, HALF), jnp.float32),
            pltpu.VMEM((C64, HALF), jnp.float32),
            pltpu.VMEM((C64, HALF), jnp.float32),
            pltpu.VMEM((C64, HALF), jnp.float32),
            pltpu.VMEM((C64, HALF), jnp.float32),
            pltpu.VMEM((BCH, C64), jnp.int32),
            pltpu.VMEM((BCH, C64), jnp.int32),
            pltpu.SemaphoreType.DMA,
            pltpu.SemaphoreType.DMA,
            pltpu.SemaphoreType.DMA,
            pltpu.SemaphoreType.DMA,
            pltpu.SemaphoreType.DMA,
            pltpu.SemaphoreType.DMA,
        ],
    )(hw2, wf2, row2, col2)


def _split_body(h_ref, o_ref):
    o_ref[0] = h_ref[:, :HALF]
    o_ref[1] = h_ref[:, HALF:]


def _tc_split(h_pad):
    return pl.pallas_call(
        _split_body,
        grid=(A_PAD // TE,),
        in_specs=[pl.BlockSpec((TE, HIDDEN), lambda t: (t, 0))],
        out_specs=pl.BlockSpec((2, TE, HALF), lambda t: (0, t, 0)),
        out_shape=jax.ShapeDtypeStruct((2, A_PAD, HALF), jnp.float32),
    )(h_pad)


def _mean_body(hb_ref, zc_ref, hbo_ref, zbo_ref):
    zc = zc_ref[...]
    lane = lax.broadcasted_iota(jnp.int32, zc.shape, 1)
    cnt = jnp.sum(jnp.where(lane == 3, zc, 0.0), axis=1, keepdims=True)
    inv = 1.0 / jnp.maximum(cnt, 1.0)
    hbo_ref[:, :HALF] = hb_ref[0] * inv
    hbo_ref[:, HALF:] = hb_ref[1] * inv
    zbo_ref[...] = jnp.where(lane == 3, 0.0, zc * inv)


def _tc_mean(hbsum, zc):
    return pl.pallas_call(
        _mean_body,
        grid=(N_BLOCKS // NT,),
        in_specs=[pl.BlockSpec((2, NT, HALF), lambda t: (0, t, 0)),
                  pl.BlockSpec((NT, LANES), lambda t: (t, 0))],
        out_specs=[pl.BlockSpec((NT, HIDDEN), lambda t: (t, 0)),
                   pl.BlockSpec((NT, LANES), lambda t: (t, 0))],
        out_shape=[jax.ShapeDtypeStruct((N_BLOCKS, HIDDEN), jnp.float32),
                   jax.ShapeDtypeStruct((N_BLOCKS, LANES), jnp.float32)],
    )(hbsum, zc)


def _wf_body(dv_ref, ea_ref, ew_ref, eb_ref, w1_ref, b1_ref, w2_ref, b2_ref,
             o0, o1, o2):
    t = pl.program_id(0)
    dv = dv_ref[...]
    dist = jnp.sqrt(jnp.sum(dv * dv, axis=1, keepdims=True) + 1e-12)
    spacing = CUTOFF / (NUM_GAUSSIANS - 1)
    coeff = -0.5 / spacing ** 2
    offs = lax.broadcasted_iota(
        jnp.int32, (TE, NUM_GAUSSIANS), 1).astype(jnp.float32) * spacing
    ef = jnp.exp(coeff * (dist - offs) ** 2)
    eap = jnp.dot(ea_ref[...], ew_ref[...],
                  preferred_element_type=jnp.float32) + eb_ref[...]
    ef = ef + eap
    cutoff = 0.5 * (jnp.cos(dist * (np.pi / CUTOFF)) + 1.0)
    cutoff = cutoff * (dist < CUTOFF).astype(jnp.float32)
    rows = lax.broadcasted_iota(jnp.int32, (TE, 1), 0) + t * TE
    scale = cutoff * (rows < N_EDGES).astype(jnp.float32)
    for l, o in enumerate((o0, o1, o2)):
        s = jnp.dot(ef, w1_ref[l], preferred_element_type=jnp.float32) + b1_ref[l]
        s = jax.nn.softplus(s) - LOG2
        wf = jnp.dot(s, w2_ref[l], preferred_element_type=jnp.float32) + b2_ref[l]
        wf = wf * scale
        o[0] = wf[:, :HALF]
        o[1] = wf[:, HALF:]


def _tc_wf(dvec, ea, ew, eb, w1s, b1s, w2s, b2s):
    full = lambda *shape: pl.BlockSpec(shape, lambda t: (0,) * len(shape))
    wf_spec = pl.BlockSpec((2, TE, HALF), lambda t: (0, t, 0))
    wf_shape = jax.ShapeDtypeStruct((2, E_PAD, HALF), jnp.float32)
    return pl.pallas_call(
        _wf_body,
        grid=(E_PAD // TE,),
        in_specs=[pl.BlockSpec((TE, LANES), lambda t: (t, 0)),
                  pl.BlockSpec((TE, EDGE_SIZE), lambda t: (t, 0)),
                  full(EDGE_SIZE, NUM_GAUSSIANS),
                  full(1, NUM_GAUSSIANS),
                  full(N_LAYERS, NUM_GAUSSIANS, HIDDEN),
                  full(N_LAYERS, HIDDEN),
                  full(N_LAYERS, HIDDEN, HIDDEN),
                  full(N_LAYERS, HIDDEN)],
        out_specs=[wf_spec, wf_spec, wf_spec],
        out_shape=[wf_shape, wf_shape, wf_shape],
    )(dvec, ea, ew, eb, w1s, b1s, w2s, b2s)


def _hw_body(h_ref, w_ref, o_ref):
    r = jnp.dot(h_ref[...], w_ref[...], preferred_element_type=jnp.float32)
    o_ref[0] = r[:, :HALF]
    o_ref[1] = r[:, HALF:]


def _tc_hw(h, w):
    return pl.pallas_call(
        _hw_body,
        grid=(N_BLOCKS // NT,),
        in_specs=[pl.BlockSpec((NT, HIDDEN), lambda t: (t, 0)),
                  pl.BlockSpec((HIDDEN, HIDDEN), lambda t: (0, 0))],
        out_specs=pl.BlockSpec((2, NT, HALF), lambda t: (0, t, 0)),
        out_shape=jax.ShapeDtypeStruct((2, N_BLOCKS, HALF), jnp.float32),
    )(h, w)


def _post_body(a_ref, h_ref, w2_ref, b2_ref, ow_ref, ob_ref, o_ref):
    x = (jnp.dot(a_ref[0], w2_ref[:HALF, :], preferred_element_type=jnp.float32)
         + jnp.dot(a_ref[1], w2_ref[HALF:, :], preferred_element_type=jnp.float32)
         + b2_ref[...])
    x = jax.nn.softplus(x) - LOG2
    x = jnp.dot(x, ow_ref[...], preferred_element_type=jnp.float32) + ob_ref[...]
    o_ref[...] = h_ref[...] + x


def _tc_post(agg, h, w2, b2, ow, ob):
    return pl.pallas_call(
        _post_body,
        grid=(N_BLOCKS // NT,),
        in_specs=[pl.BlockSpec((2, NT, HALF), lambda t: (0, t, 0)),
                  pl.BlockSpec((NT, HIDDEN), lambda t: (t, 0)),
                  pl.BlockSpec((HIDDEN, HIDDEN), lambda t: (0, 0)),
                  pl.BlockSpec((1, HIDDEN), lambda t: (0, 0)),
                  pl.BlockSpec((HIDDEN, HIDDEN), lambda t: (0, 0)),
                  pl.BlockSpec((1, HIDDEN), lambda t: (0, 0))],
        out_specs=pl.BlockSpec((NT, HIDDEN), lambda t: (t, 0)),
        out_shape=jax.ShapeDtypeStruct((N_BLOCKS, HIDDEN), jnp.float32),
    )(agg, h, w2, b2, ow, ob)


def _norm_body(x_ref, o_ref):
    x = x_ref[...]
    n = jnp.sqrt(jnp.sum(x * x, axis=1, keepdims=True))
    o_ref[...] = x / jnp.maximum(n, 1e-12)


def _tc_norm(x, tile):
    n = x.shape[0]
    return pl.pallas_call(
        _norm_body,
        grid=(n // tile,),
        in_specs=[pl.BlockSpec((tile, HIDDEN), lambda t: (t, 0))],
        out_specs=pl.BlockSpec((tile, HIDDEN), lambda t: (t, 0)),
        out_shape=jax.ShapeDtypeStruct((n, HIDDEN), jnp.float32),
    )(x)


def _pool_body(bid_ref, br_ref, o_ref):
    t = pl.program_id(0)

    @pl.when(t == 0)
    def _():
        o_ref[...] = jnp.zeros_like(o_ref)

    bid = bid_ref[0, 0, :]
    gi = lax.broadcasted_iota(jnp.int32, (N_GRAPHS, NT), 0)
    oh = (gi == bid[None, :]).astype(jnp.float32)
    o_ref[...] += jnp.dot(oh, br_ref[...], preferred_element_type=jnp.float32)


def _tc_pool(bid3, br):
    return pl.pallas_call(
        _pool_body,
        grid=(N_BLOCKS // NT,),
        in_specs=[pl.BlockSpec((1, 1, NT), lambda t: (t, 0, 0)),
                  pl.BlockSpec((NT, HIDDEN), lambda t: (t, 0))],
        out_specs=pl.BlockSpec((N_GRAPHS, HIDDEN), lambda t: (0, 0)),
        out_shape=jax.ShapeDtypeStruct((N_GRAPHS, HIDDEN), jnp.float32),
    )(bid3, br)


def kernel(H, Z, block_id, batch_id, edges, edge_attr, params):
    f32 = jnp.float32
    H = H.astype(f32)
    Z = Z.astype(f32)
    edge_attr = edge_attr.astype(f32)
    bid_p = jnp.pad(block_id.astype(jnp.int32), (0, A_PAD - N_ATOMS))
    rowp = jnp.pad(edges[0].astype(jnp.int32), (0, E_PAD - N_EDGES))
    colp = jnp.pad(edges[1].astype(jnp.int32), (0, E_PAD - N_EDGES))

    h_pad = jnp.pad(H, ((0, A_PAD - N_ATOMS), (0, 0)))
    zlane = jnp.concatenate(
        [Z, jnp.ones((N_ATOMS, 1), f32), jnp.zeros((N_ATOMS, 12), f32)], axis=1)
    zp = jnp.pad(zlane, ((0, A_PAD - N_ATOMS), (0, 0)))
    ea_p = jnp.pad(edge_attr, ((0, E_PAD - N_EDGES), (0, 0)))

    hsplit = _tc_split(h_pad).reshape(2 * A_PAD, HALF)
    hbsum2, zc2 = _sc_atoms_call(hsplit, zp, bid_p)
    hb, zb16 = _tc_mean(hbsum2.reshape(2, N_BLOCKS, HALF), zc2[:N_BLOCKS])

    dvec = _sc_dvec_call(zb16, rowp, colp)

    lp = params['layers']
    w1s = jnp.stack([l['mlp_w1'] for l in lp]).astype(f32)
    b1s = jnp.stack([l['mlp_b1'] for l in lp]).astype(f32)
    w2s = jnp.stack([l['mlp_w2'] for l in lp]).astype(f32)
    b2s = jnp.stack([l['mlp_b2'] for l in lp]).astype(f32)
    wfs = _tc_wf(dvec, ea_p,
                 params['edge_linear_w'].astype(f32),
                 params['edge_linear_b'].astype(f32).reshape(1, NUM_GAUSSIANS),
                 w1s, b1s, w2s, b2s)

    h = hb
    for l in range(N_LAYERS):
        hw2 = _tc_hw(h, lp[l]['conv_lin1_w'].astype(f32))
        agg2 = _sc_edge_call(hw2.reshape(2 * N_BLOCKS, HALF),
                             wfs[l].reshape(2 * E_PAD, HALF),
                             rowp.reshape(E_PAD // C64, C64),
                             colp.reshape(E_PAD // C64, C64))
        h = _tc_post(agg2.reshape(2, N_BLOCKS, HALF), h,
                     lp[l]['conv_lin2_w'].astype(f32),
                     lp[l]['conv_lin2_b'].astype(f32).reshape(1, HIDDEN),
                     lp[l]['out_w'].astype(f32),
                     lp[l]['out_b'].astype(f32).reshape(1, HIDDEN))

    block_repr = _tc_norm(h, NT)
    graph = _tc_pool(batch_id.astype(jnp.int32).reshape(N_BLOCKS // NT, 1, NT),
                     block_repr)
    graph_repr = _tc_norm(graph, N_GRAPHS)
    return (hb, block_repr, graph_repr, None)


# speedup vs baseline: 1.8334x; 1.0612x over previous
import functools

import numpy as np
import jax
import jax.numpy as jnp
from jax import lax
from jax.experimental import pallas as pl
from jax.experimental.pallas import tpu as pltpu
from jax.experimental.pallas import tpu_sc as plsc

N_ATOMS = 50000
N_BLOCKS = 10000
N_GRAPHS = 64
N_EDGES = 160000
HIDDEN = 256
EDGE_SIZE = 16
NUM_GAUSSIANS = 50
N_LAYERS = 3
CUTOFF = 10.0
LOG2 = float(np.log(2.0))

NC = 2
NS = 16
LANES = 16
HALF = HIDDEN // 2
CHUNK = 128
A_PAD = 51200
E_PAD = 163840
ROWS_B = N_BLOCKS // NS
NT = 1000
TE = 512

@functools.cache
def _mesh():
    return plsc.VectorSubcoreMesh(
        core_axis_name="c", subcore_axis_name="s", num_cores=NC, num_subcores=NS)


def _zero_shared_slice(buf, rows, width_groups, shared, sid):
    def zl(i, _):
        for j in range(width_groups):
            buf[i, pl.ds(j * LANES, LANES)] = jnp.zeros((LANES,), jnp.float32)
        return 0
    lax.fori_loop(0, rows, zl, 0)
    full, rem = ROWS_B // rows, ROWS_B % rows
    for k in range(full):
        pltpu.sync_copy(buf.at[pl.ds(0, rows)],
                        shared.at[pl.ds(sid * ROWS_B + k * rows, rows)])
    if rem:
        pltpu.sync_copy(buf.at[pl.ds(0, rem)],
                        shared.at[pl.ds(sid * ROWS_B + full * rows, rem)])


def _sc_atoms_body(hs_ref, zp_ref, bid_ref, hbsum_ref, zc_ref,
                   hsh, zsh, vh, vz, idxv, sem):
    del sem
    cid = lax.axis_index("c")
    sid = lax.axis_index("s")
    _zero_shared_slice(vh, CHUNK, HALF // LANES, hsh, sid)
    _zero_shared_slice(vz, CHUNK, 1, zsh, sid)
    plsc.subcore_barrier()
    tile_rows = A_PAD // NS
    tile_base = sid * tile_rows

    def chunk(k, _):
        base = tile_base + k * CHUNK
        pltpu.sync_copy(bid_ref.at[pl.ds(base, CHUNK)], idxv)
        pltpu.sync_copy(hs_ref.at[pl.ds(cid * A_PAD + base, CHUNK)], vh)
        pltpu.sync_copy(zp_ref.at[pl.ds(base, CHUNK)], vz)
        pltpu.sync_copy(vh, hsh.at[idxv], add=True)
        pltpu.sync_copy(vz, zsh.at[idxv], add=True)
        return 0

    lax.fori_loop(0, tile_rows // CHUNK, chunk, 0)
    plsc.subcore_barrier()
    r0 = sid * ROWS_B
    out0 = cid * N_BLOCKS + r0
    pltpu.sync_copy(hsh.at[pl.ds(r0, ROWS_B)], hbsum_ref.at[pl.ds(out0, ROWS_B)])
    pltpu.sync_copy(zsh.at[pl.ds(r0, ROWS_B)], zc_ref.at[pl.ds(out0, ROWS_B)])


def _sc_atoms_call(hsplit, zp, bid):
    return pl.kernel(
        _sc_atoms_body,
        out_type=(jax.ShapeDtypeStruct((NC * N_BLOCKS, HALF), jnp.float32),
                  jax.ShapeDtypeStruct((NC * N_BLOCKS, LANES), jnp.float32)),
        mesh=_mesh(),
        compiler_params=pltpu.CompilerParams(use_tc_tiling_on_sc=False),
        scratch_types=[
            pltpu.VMEM_SHARED((N_BLOCKS, HALF), jnp.float32),
            pltpu.VMEM_SHARED((N_BLOCKS, LANES), jnp.float32),
            pltpu.VMEM((CHUNK, HALF), jnp.float32),
            pltpu.VMEM((CHUNK, LANES), jnp.float32),
            pltpu.VMEM((CHUNK,), jnp.int32),
            pltpu.SemaphoreType.DMA,
        ],
    )(hsplit, zp, bid)


def _sc_dvec_body(zb_ref, row_ref, col_ref, dvec_ref,
                  zr, zc, db, rv, cv, sem):
    cid = lax.axis_index("c")
    sid = lax.axis_index("s")
    w = sid * NC + cid
    per_w = E_PAD // (NC * NS)

    def chunk(k, _):
        base = w * per_w + k * CHUNK
        pltpu.sync_copy(row_ref.at[pl.ds(base, CHUNK)], rv)
        pltpu.sync_copy(col_ref.at[pl.ds(base, CHUNK)], cv)
        pltpu.async_copy(zb_ref.at[rv], zr, sem).wait()
        pltpu.async_copy(zb_ref.at[cv], zc, sem).wait()

        def sub(i, _):
            s = pl.ds(0, LANES)
            db[i, s] = zr[i, s] - zc[i, s]
            return 0

        lax.fori_loop(0, CHUNK, sub, 0)
        pltpu.sync_copy(db, dvec_ref.at[pl.ds(base, CHUNK)])
        return 0

    lax.fori_loop(0, per_w // CHUNK, chunk, 0)


def _sc_dvec_call(zb16, rowp, colp):
    return pl.kernel(
        _sc_dvec_body,
        out_type=jax.ShapeDtypeStruct((E_PAD, LANES), jnp.float32),
        mesh=_mesh(),
        compiler_params=pltpu.CompilerParams(use_tc_tiling_on_sc=False),
        scratch_types=[
            pltpu.VMEM((CHUNK, LANES), jnp.float32),
            pltpu.VMEM((CHUNK, LANES), jnp.float32),
            pltpu.VMEM((CHUNK, LANES), jnp.float32),
            pltpu.VMEM((CHUNK,), jnp.int32),
            pltpu.VMEM((CHUNK,), jnp.int32),
            pltpu.SemaphoreType.DMA,
        ],
    )(zb16, rowp, colp)


C64 = 64
BCH = 80
NBATCH = E_PAD // NS // C64 // BCH


def _sc_edge_body(hw_ref, wf_ref, row2_ref, col2_ref, agg_ref,
                  ash, xb0, xb1, wb0, wb1, rvb, cvb,
                  sg0, sg1, sw0, sw1, ss0, ss1):
    cid = lax.axis_index("c")
    sid = lax.axis_index("s")
    _zero_shared_slice(xb0, C64, HALF // LANES, ash, sid)
    plsc.subcore_barrier()
    off = cid * N_BLOCKS
    wf_row0 = cid * (E_PAD // C64)
    xb = (xb0, xb1)
    wb = (wb0, wb1)
    sg = (sg0, sg1)
    sw = (sw0, sw1)
    ss = (ss0, ss1)

    def mul(b):
        def ml(i, _):
            for q in range(HALF // LANES):
                s = pl.ds(q * LANES, LANES)
                xb[b][i, s] = xb[b][i, s] * wb[b][i, s]
            return 0
        lax.fori_loop(0, C64, ml, 0)

    for bb in range(NBATCH):
        g0 = sid * (NBATCH * BCH) + bb * BCH
        pltpu.sync_copy(row2_ref.at[pl.ds(g0, BCH)], rvb)
        pltpu.sync_copy(col2_ref.at[pl.ds(g0, BCH)], cvb)

        def addoff(r, _):
            for q in range(C64 // LANES):
                s = pl.ds(q * LANES, LANES)
                cvb[r, s] = cvb[r, s] + off
            return 0

        lax.fori_loop(0, BCH, addoff, 0)

        def gather(j, b):
            pltpu.async_copy(hw_ref.at[cvb.at[j]], xb[b], sg[b])
            pltpu.async_copy(wf_ref.at[pl.ds((wf_row0 + g0 + j) * C64, C64)],
                             wb[b], sw[b])

        def wait_gw(j, b):
            pltpu.make_async_copy(hw_ref.at[cvb.at[j]], xb[b], sg[b]).wait()
            pltpu.make_async_copy(
                wf_ref.at[pl.ds((wf_row0 + g0 + j) * C64, C64)],
                wb[b], sw[b]).wait()

        def scat(j, b):
            pltpu.async_copy(xb[b], ash.at[rvb.at[j]], ss[b], add=True)

        def wait_s(j, b):
            pltpu.make_async_copy(xb[b], ash.at[rvb.at[j]], ss[b]).wait()

        gather(0, 0)
        gather(1, 1)

        def step(kk, _):
            j0 = 2 * kk
            wait_gw(j0, 0)
            mul(0)
            scat(j0, 0)
            wait_s(j0, 0)
            gather(j0 + 2, 0)
            wait_gw(j0 + 1, 1)
            mul(1)
            scat(j0 + 1, 1)
            wait_s(j0 + 1, 1)
            gather(j0 + 3, 1)
            return 0

        lax.fori_loop(0, BCH // 2 - 1, step, 0)
        wait_gw(BCH - 2, 0)
        mul(0)
        scat(BCH - 2, 0)
        wait_gw(BCH - 1, 1)
        mul(1)
        scat(BCH - 1, 1)
        wait_s(BCH - 2, 0)
        wait_s(BCH - 1, 1)

    plsc.subcore_barrier()
    r0 = sid * ROWS_B
    pltpu.sync_copy(ash.at[pl.ds(r0, ROWS_B)],
                    agg_ref.at[pl.ds(cid * N_BLOCKS + r0, ROWS_B)])


def _sc_edge_call(hw2, wf2, row2, col2):
    return pl.kernel(
        _sc_edge_body,
        out_type=jax.ShapeDtypeStruct((NC * N_BLOCKS, HALF), jnp.float32),
        mesh=_mesh(),
        compiler_params=pltpu.CompilerParams(use_tc_tiling_on_sc=False),
        scratch_types=[
            pltpu.VMEM_SHARED((N_BLOCKS, HALF), jnp.float32),
            pltpu.VMEM((C64, HALF), jnp.float32),
            pltpu.VMEM((C64, HALF), jnp.float32),
            pltpu.VMEM((C64, HALF), jnp.float32),
            pltpu.VMEM((C64, HALF), jnp.float32),
            pltpu.VMEM((BCH, C64), jnp.int32),
            pltpu.VMEM((BCH, C64), jnp.int32),
            pltpu.SemaphoreType.DMA,
            pltpu.SemaphoreType.DMA,
            pltpu.SemaphoreType.DMA,
            pltpu.SemaphoreType.DMA,
            pltpu.SemaphoreType.DMA,
            pltpu.SemaphoreType.DMA,
        ],
    )(hw2, wf2, row2, col2)


def _split_body(h_ref, o_ref):
    o_ref[0] = h_ref[:, :HALF]
    o_ref[1] = h_ref[:, HALF:]


def _tc_split(h_pad):
    return pl.pallas_call(
        _split_body,
        grid=(A_PAD // TE,),
        in_specs=[pl.BlockSpec((TE, HIDDEN), lambda t: (t, 0))],
        out_specs=pl.BlockSpec((2, TE, HALF), lambda t: (0, t, 0)),
        out_shape=jax.ShapeDtypeStruct((2, A_PAD, HALF), jnp.float32),
    )(h_pad)


def _mean_body(hb_ref, zc_ref, hbo_ref, zbo_ref):
    zc = zc_ref[...]
    lane = lax.broadcasted_iota(jnp.int32, zc.shape, 1)
    cnt = jnp.sum(jnp.where(lane == 3, zc, 0.0), axis=1, keepdims=True)
    inv = 1.0 / jnp.maximum(cnt, 1.0)
    hbo_ref[:, :HALF] = hb_ref[0] * inv
    hbo_ref[:, HALF:] = hb_ref[1] * inv
    zbo_ref[...] = jnp.where(lane == 3, 0.0, zc * inv)


def _tc_mean(hbsum, zc):
    return pl.pallas_call(
        _mean_body,
        grid=(N_BLOCKS // NT,),
        in_specs=[pl.BlockSpec((2, NT, HALF), lambda t: (0, t, 0)),
                  pl.BlockSpec((NT, LANES), lambda t: (t, 0))],
        out_specs=[pl.BlockSpec((NT, HIDDEN), lambda t: (t, 0)),
                   pl.BlockSpec((NT, LANES), lambda t: (t, 0))],
        out_shape=[jax.ShapeDtypeStruct((N_BLOCKS, HIDDEN), jnp.float32),
                   jax.ShapeDtypeStruct((N_BLOCKS, LANES), jnp.float32)],
    )(hbsum, zc)


def _wf_body(dv_ref, ea_ref, ew_ref, eb_ref, w1_ref, b1_ref, w2_ref, b2_ref,
             o0, o1, o2):
    t = pl.program_id(0)
    dv = dv_ref[...]
    dist = jnp.sqrt(jnp.sum(dv * dv, axis=1, keepdims=True) + 1e-12)
    spacing = CUTOFF / (NUM_GAUSSIANS - 1)
    coeff = -0.5 / spacing ** 2
    offs = lax.broadcasted_iota(
        jnp.int32, (TE, NUM_GAUSSIANS), 1).astype(jnp.float32) * spacing
    ef = jnp.exp(coeff * (dist - offs) ** 2)
    eap = jnp.dot(ea_ref[...], ew_ref[...],
                  preferred_element_type=jnp.float32) + eb_ref[...]
    ef = ef + eap
    cutoff = 0.5 * (jnp.cos(dist * (np.pi / CUTOFF)) + 1.0)
    cutoff = cutoff * (dist < CUTOFF).astype(jnp.float32)
    rows = lax.broadcasted_iota(jnp.int32, (TE, 1), 0) + t * TE
    scale = cutoff * (rows < N_EDGES).astype(jnp.float32)
    for l, o in enumerate((o0, o1, o2)):
        s = jnp.dot(ef, w1_ref[l], preferred_element_type=jnp.float32) + b1_ref[l]
        s = jax.nn.softplus(s) - LOG2
        wf = jnp.dot(s, w2_ref[l], preferred_element_type=jnp.float32) + b2_ref[l]
        wf = wf * scale
        o[0] = wf[:, :HALF]
        o[1] = wf[:, HALF:]


def _tc_wf(dvec, ea, ew, eb, w1s, b1s, w2s, b2s):
    full = lambda *shape: pl.BlockSpec(shape, lambda t: (0,) * len(shape))
    wf_spec = pl.BlockSpec((2, TE, HALF), lambda t: (0, t, 0))
    wf_shape = jax.ShapeDtypeStruct((2, E_PAD, HALF), jnp.float32)
    return pl.pallas_call(
        _wf_body,
        grid=(E_PAD // TE,),
        in_specs=[pl.BlockSpec((TE, LANES), lambda t: (t, 0)),
                  pl.BlockSpec((TE, EDGE_SIZE), lambda t: (t, 0)),
                  full(EDGE_SIZE, NUM_GAUSSIANS),
                  full(1, NUM_GAUSSIANS),
                  full(N_LAYERS, NUM_GAUSSIANS, HIDDEN),
                  full(N_LAYERS, HIDDEN),
                  full(N_LAYERS, HIDDEN, HIDDEN),
                  full(N_LAYERS, HIDDEN)],
        out_specs=[wf_spec, wf_spec, wf_spec],
        out_shape=[wf_shape, wf_shape, wf_shape],
    )(dvec, ea, ew, eb, w1s, b1s, w2s, b2s)


def _hw_body(h_ref, w_ref, o_ref):
    r = jnp.dot(h_ref[...], w_ref[...], preferred_element_type=jnp.float32)
    o_ref[0] = r[:, :HALF]
    o_ref[1] = r[:, HALF:]


def _tc_hw(h, w):
    return pl.pallas_call(
        _hw_body,
        grid=(N_BLOCKS // NT,),
        in_specs=[pl.BlockSpec((NT, HIDDEN), lambda t: (t, 0)),
                  pl.BlockSpec((HIDDEN, HIDDEN), lambda t: (0, 0))],
        out_specs=pl.BlockSpec((2, NT, HALF), lambda t: (0, t, 0)),
        out_shape=jax.ShapeDtypeStruct((2, N_BLOCKS, HALF), jnp.float32),
    )(h, w)


def _post_body(a_ref, h_ref, w2_ref, b2_ref, ow_ref, ob_ref, o_ref):
    x = (jnp.dot(a_ref[0], w2_ref[:HALF, :], preferred_element_type=jnp.float32)
         + jnp.dot(a_ref[1], w2_ref[HALF:, :], preferred_element_type=jnp.float32)
         + b2_ref[...])
    x = jax.nn.softplus(x) - LOG2
    x = jnp.dot(x, ow_ref[...], preferred_element_type=jnp.float32) + ob_ref[...]
    o_ref[...] = h_ref[...] + x


def _tc_post(agg, h, w2, b2, ow, ob):
    return pl.pallas_call(
        _post_body,
        grid=(N_BLOCKS // NT,),
        in_specs=[pl.BlockSpec((2, NT, HALF), lambda t: (0, t, 0)),
                  pl.BlockSpec((NT, HIDDEN), lambda t: (t, 0)),
                  pl.BlockSpec((HIDDEN, HIDDEN), lambda t: (0, 0)),
                  pl.BlockSpec((1, HIDDEN), lambda t: (0, 0)),
                  pl.BlockSpec((HIDDEN, HIDDEN), lambda t: (0, 0)),
                  pl.BlockSpec((1, HIDDEN), lambda t: (0, 0))],
        out_specs=pl.BlockSpec((NT, HIDDEN), lambda t: (t, 0)),
        out_shape=jax.ShapeDtypeStruct((N_BLOCKS, HIDDEN), jnp.float32),
    )(agg, h, w2, b2, ow, ob)


def _norm_body(x_ref, o_ref):
    x = x_ref[...]
    n = jnp.sqrt(jnp.sum(x * x, axis=1, keepdims=True))
    o_ref[...] = x / jnp.maximum(n, 1e-12)


def _tc_norm(x, tile):
    n = x.shape[0]
    return pl.pallas_call(
        _norm_body,
        grid=(n // tile,),
        in_specs=[pl.BlockSpec((tile, HIDDEN), lambda t: (t, 0))],
        out_specs=pl.BlockSpec((tile, HIDDEN), lambda t: (t, 0)),
        out_shape=jax.ShapeDtypeStruct((n, HIDDEN), jnp.float32),
    )(x)


def _pool_body(bid_ref, br_ref, o_ref):
    t = pl.program_id(0)

    @pl.when(t == 0)
    def _():
        o_ref[...] = jnp.zeros_like(o_ref)

    bid = bid_ref[0, 0, :]
    gi = lax.broadcasted_iota(jnp.int32, (N_GRAPHS, NT), 0)
    oh = (gi == bid[None, :]).astype(jnp.float32)
    o_ref[...] += jnp.dot(oh, br_ref[...], preferred_element_type=jnp.float32)


def _tc_pool(bid3, br):
    return pl.pallas_call(
        _pool_body,
        grid=(N_BLOCKS // NT,),
        in_specs=[pl.BlockSpec((1, 1, NT), lambda t: (t, 0, 0)),
                  pl.BlockSpec((NT, HIDDEN), lambda t: (t, 0))],
        out_specs=pl.BlockSpec((N_GRAPHS, HIDDEN), lambda t: (0, 0)),
        out_shape=jax.ShapeDtypeStruct((N_GRAPHS, HIDDEN), jnp.float32),
    )(bid3, br)


def kernel(H, Z, block_id, batch_id, edges, edge_attr, params):
    f32 = jnp.float32
    H = H.astype(f32)
    Z = Z.astype(f32)
    edge_attr = edge_attr.astype(f32)
    bid_p = jnp.pad(block_id.astype(jnp.int32), (0, A_PAD - N_ATOMS))
    rowp = jnp.pad(edges[0].astype(jnp.int32), (0, E_PAD - N_EDGES))
    colp = jnp.pad(edges[1].astype(jnp.int32), (0, E_PAD - N_EDGES))

    h_pad = jnp.pad(H, ((0, A_PAD - N_ATOMS), (0, 0)))
    zlane = jnp.concatenate(
        [Z, jnp.ones((N_ATOMS, 1), f32), jnp.zeros((N_ATOMS, 12), f32)], axis=1)
    zp = jnp.pad(zlane, ((0, A_PAD - N_ATOMS), (0, 0)))
    ea_p = jnp.pad(edge_attr, ((0, E_PAD - N_EDGES), (0, 0)))

    hsplit = _tc_split(h_pad).reshape(2 * A_PAD, HALF)
    hbsum2, zc2 = _sc_atoms_call(hsplit, zp, bid_p)
    hb, zb16 = _tc_mean(hbsum2.reshape(2, N_BLOCKS, HALF), zc2[:N_BLOCKS])

    dvec = _sc_dvec_call(zb16, rowp, colp)

    lp = params['layers']
    w1s = jnp.stack([l['mlp_w1'] for l in lp]).astype(f32)
    b1s = jnp.stack([l['mlp_b1'] for l in lp]).astype(f32)
    w2s = jnp.stack([l['mlp_w2'] for l in lp]).astype(f32)
    b2s = jnp.stack([l['mlp_b2'] for l in lp]).astype(f32)
    wfs = _tc_wf(dvec, ea_p,
                 params['edge_linear_w'].astype(f32),
                 params['edge_linear_b'].astype(f32).reshape(1, NUM_GAUSSIANS),
                 w1s, b1s, w2s, b2s)

    h = hb
    for l in range(N_LAYERS):
        hw2 = _tc_hw(h, lp[l]['conv_lin1_w'].astype(f32))
        agg2 = _sc_edge_call(hw2.reshape(2 * N_BLOCKS, HALF),
                             wfs[l].reshape(2 * E_PAD, HALF),
                             rowp.reshape(E_PAD // C64, C64),
                             colp.reshape(E_PAD // C64, C64))
        h = _tc_post(agg2.reshape(2, N_BLOCKS, HALF), h,
                     lp[l]['conv_lin2_w'].astype(f32),
                     lp[l]['conv_lin2_b'].astype(f32).reshape(1, HIDDEN),
                     lp[l]['out_w'].astype(f32),
                     lp[l]['out_b'].astype(f32).reshape(1, HIDDEN))

    block_repr = _tc_norm(h, NT)
    graph = _tc_pool(batch_id.astype(jnp.int32).reshape(N_BLOCKS // NT, 1, NT),
                     block_repr)
    graph_repr = _tc_norm(graph, N_GRAPHS)
    return (hb, block_repr, graph_repr, None)


# speedup vs baseline: 1.8547x; 1.0116x over previous
import functools

import numpy as np
import jax
import jax.numpy as jnp
from jax import lax
from jax.experimental import pallas as pl
from jax.experimental.pallas import tpu as pltpu
from jax.experimental.pallas import tpu_sc as plsc

N_ATOMS = 50000
N_BLOCKS = 10000
N_GRAPHS = 64
N_EDGES = 160000
HIDDEN = 256
EDGE_SIZE = 16
NUM_GAUSSIANS = 50
N_LAYERS = 3
CUTOFF = 10.0
LOG2 = float(np.log(2.0))

NC = 2
NS = 16
LANES = 16
HALF = HIDDEN // 2
CHUNK = 128
A_PAD = 51200
E_PAD = 163840
ROWS_B = N_BLOCKS // NS
NT = 1000
TE = 512

@functools.cache
def _mesh():
    return plsc.VectorSubcoreMesh(
        core_axis_name="c", subcore_axis_name="s", num_cores=NC, num_subcores=NS)


def _zero_shared_slice(buf, rows, width_groups, shared, sid):
    def zl(i, _):
        for j in range(width_groups):
            buf[i, pl.ds(j * LANES, LANES)] = jnp.zeros((LANES,), jnp.float32)
        return 0
    lax.fori_loop(0, rows, zl, 0)
    full, rem = ROWS_B // rows, ROWS_B % rows
    for k in range(full):
        pltpu.sync_copy(buf.at[pl.ds(0, rows)],
                        shared.at[pl.ds(sid * ROWS_B + k * rows, rows)])
    if rem:
        pltpu.sync_copy(buf.at[pl.ds(0, rem)],
                        shared.at[pl.ds(sid * ROWS_B + full * rows, rem)])


def _sc_atoms_body(hs_ref, zp_ref, bid_ref, hbsum_ref, zc_ref,
                   hsh, zsh, vh, vz, idxv, sem):
    del sem
    cid = lax.axis_index("c")
    sid = lax.axis_index("s")
    _zero_shared_slice(vh, CHUNK, HALF // LANES, hsh, sid)
    _zero_shared_slice(vz, CHUNK, 1, zsh, sid)
    plsc.subcore_barrier()
    tile_rows = A_PAD // NS
    tile_base = sid * tile_rows

    def chunk(k, _):
        base = tile_base + k * CHUNK
        pltpu.sync_copy(bid_ref.at[pl.ds(base, CHUNK)], idxv)
        pltpu.sync_copy(hs_ref.at[pl.ds(cid * A_PAD + base, CHUNK)], vh)
        pltpu.sync_copy(zp_ref.at[pl.ds(base, CHUNK)], vz)
        pltpu.sync_copy(vh, hsh.at[idxv], add=True)
        pltpu.sync_copy(vz, zsh.at[idxv], add=True)
        return 0

    lax.fori_loop(0, tile_rows // CHUNK, chunk, 0)
    plsc.subcore_barrier()
    r0 = sid * ROWS_B
    out0 = cid * N_BLOCKS + r0
    pltpu.sync_copy(hsh.at[pl.ds(r0, ROWS_B)], hbsum_ref.at[pl.ds(out0, ROWS_B)])
    pltpu.sync_copy(zsh.at[pl.ds(r0, ROWS_B)], zc_ref.at[pl.ds(out0, ROWS_B)])


def _sc_atoms_call(hsplit, zp, bid):
    return pl.kernel(
        _sc_atoms_body,
        out_type=(jax.ShapeDtypeStruct((NC * N_BLOCKS, HALF), jnp.float32),
                  jax.ShapeDtypeStruct((NC * N_BLOCKS, LANES), jnp.float32)),
        mesh=_mesh(),
        compiler_params=pltpu.CompilerParams(use_tc_tiling_on_sc=False),
        scratch_types=[
            pltpu.VMEM_SHARED((N_BLOCKS, HALF), jnp.float32),
            pltpu.VMEM_SHARED((N_BLOCKS, LANES), jnp.float32),
            pltpu.VMEM((CHUNK, HALF), jnp.float32),
            pltpu.VMEM((CHUNK, LANES), jnp.float32),
            pltpu.VMEM((CHUNK,), jnp.int32),
            pltpu.SemaphoreType.DMA,
        ],
    )(hsplit, zp, bid)


def _sc_dvec_body(zb_ref, row_ref, col_ref, dvec_ref,
                  zr, zc, db, rv, cv, sem):
    cid = lax.axis_index("c")
    sid = lax.axis_index("s")
    w = sid * NC + cid
    per_w = E_PAD // (NC * NS)

    def chunk(k, _):
        base = w * per_w + k * CHUNK
        pltpu.sync_copy(row_ref.at[pl.ds(base, CHUNK)], rv)
        pltpu.sync_copy(col_ref.at[pl.ds(base, CHUNK)], cv)
        pltpu.async_copy(zb_ref.at[rv], zr, sem).wait()
        pltpu.async_copy(zb_ref.at[cv], zc, sem).wait()

        def sub(i, _):
            s = pl.ds(0, LANES)
            db[i, s] = zr[i, s] - zc[i, s]
            return 0

        lax.fori_loop(0, CHUNK, sub, 0)
        pltpu.sync_copy(db, dvec_ref.at[pl.ds(base, CHUNK)])
        return 0

    lax.fori_loop(0, per_w // CHUNK, chunk, 0)


def _sc_dvec_call(zb16, rowp, colp):
    return pl.kernel(
        _sc_dvec_body,
        out_type=jax.ShapeDtypeStruct((E_PAD, LANES), jnp.float32),
        mesh=_mesh(),
        compiler_params=pltpu.CompilerParams(use_tc_tiling_on_sc=False),
        scratch_types=[
            pltpu.VMEM((CHUNK, LANES), jnp.float32),
            pltpu.VMEM((CHUNK, LANES), jnp.float32),
            pltpu.VMEM((CHUNK, LANES), jnp.float32),
            pltpu.VMEM((CHUNK,), jnp.int32),
            pltpu.VMEM((CHUNK,), jnp.int32),
            pltpu.SemaphoreType.DMA,
        ],
    )(zb16, rowp, colp)


C32 = 32
BCH = 160
NBATCH = E_PAD // NS // C32 // BCH
NBUF = 4


def _sc_edge_body(hw_ref, wf_ref, row2_ref, col2_ref, agg_ref,
                  ash, xb0, xb1, xb2, xb3, wb0, wb1, wb2, wb3, rvb, cvb,
                  sg0, sg1, sg2, sg3, sw0, sw1, sw2, sw3,
                  ss0, ss1, ss2, ss3):
    cid = lax.axis_index("c")
    sid = lax.axis_index("s")
    _zero_shared_slice(xb0, C32, HALF // LANES, ash, sid)
    plsc.subcore_barrier()
    off = cid * N_BLOCKS
    wf_row0 = cid * (E_PAD // C32)
    xb = (xb0, xb1, xb2, xb3)
    wb = (wb0, wb1, wb2, wb3)
    sg = (sg0, sg1, sg2, sg3)
    sw = (sw0, sw1, sw2, sw3)
    ss = (ss0, ss1, ss2, ss3)

    def mul(b):
        def ml(i, _):
            for q in range(HALF // LANES):
                s = pl.ds(q * LANES, LANES)
                xb[b][i, s] = xb[b][i, s] * wb[b][i, s]
            return 0
        lax.fori_loop(0, C32, ml, 0)

    for bb in range(NBATCH):
        g0 = sid * (NBATCH * BCH) + bb * BCH
        pltpu.sync_copy(row2_ref.at[pl.ds(g0, BCH)], rvb)
        pltpu.sync_copy(col2_ref.at[pl.ds(g0, BCH)], cvb)

        def addoff(r, _):
            for q in range(C32 // LANES):
                s = pl.ds(q * LANES, LANES)
                cvb[r, s] = cvb[r, s] + off
            return 0

        lax.fori_loop(0, BCH, addoff, 0)

        def gather(j, b):
            pltpu.async_copy(hw_ref.at[cvb.at[j]], xb[b], sg[b])
            pltpu.async_copy(wf_ref.at[pl.ds((wf_row0 + g0 + j) * C32, C32)],
                             wb[b], sw[b])

        def wait_gw(j, b):
            pltpu.make_async_copy(hw_ref.at[cvb.at[j]], xb[b], sg[b]).wait()
            pltpu.make_async_copy(
                wf_ref.at[pl.ds((wf_row0 + g0 + j) * C32, C32)],
                wb[b], sw[b]).wait()

        def scat(j, b):
            pltpu.async_copy(xb[b], ash.at[rvb.at[j]], ss[b], add=True)

        def wait_s(j, b):
            pltpu.make_async_copy(xb[b], ash.at[rvb.at[j]], ss[b]).wait()

        for b in range(NBUF):
            gather(b, b)

        def step(kk, _):
            j0 = NBUF * kk
            for b in range(NBUF):
                j = j0 + b
                wait_gw(j, b)
                mul(b)
                scat(j, b)
                wait_s(j, b)
                gather(j + NBUF, b)
            return 0

        lax.fori_loop(0, BCH // NBUF - 1, step, 0)
        for b in range(NBUF):
            j = BCH - NBUF + b
            wait_gw(j, b)
            mul(b)
            scat(j, b)
            wait_s(j, b)

    plsc.subcore_barrier()
    r0 = sid * ROWS_B
    pltpu.sync_copy(ash.at[pl.ds(r0, ROWS_B)],
                    agg_ref.at[pl.ds(cid * N_BLOCKS + r0, ROWS_B)])


def _sc_edge_call(hw2, wf2, row2, col2):
    return pl.kernel(
        _sc_edge_body,
        out_type=jax.ShapeDtypeStruct((NC * N_BLOCKS, HALF), jnp.float32),
        mesh=_mesh(),
        compiler_params=pltpu.CompilerParams(use_tc_tiling_on_sc=False),
        scratch_types=(
            [pltpu.VMEM_SHARED((N_BLOCKS, HALF), jnp.float32)]
            + [pltpu.VMEM((C32, HALF), jnp.float32) for _ in range(8)]
            + [pltpu.VMEM((BCH, C32), jnp.int32) for _ in range(2)]
            + [pltpu.SemaphoreType.DMA for _ in range(12)]
        ),
    )(hw2, wf2, row2, col2)


def _split_body(h_ref, o_ref):
    o_ref[0] = h_ref[:, :HALF]
    o_ref[1] = h_ref[:, HALF:]


def _tc_split(h_pad):
    return pl.pallas_call(
        _split_body,
        grid=(A_PAD // TE,),
        in_specs=[pl.BlockSpec((TE, HIDDEN), lambda t: (t, 0))],
        out_specs=pl.BlockSpec((2, TE, HALF), lambda t: (0, t, 0)),
        out_shape=jax.ShapeDtypeStruct((2, A_PAD, HALF), jnp.float32),
    )(h_pad)


def _mean_body(hb_ref, zc_ref, hbo_ref, zbo_ref):
    zc = zc_ref[...]
    lane = lax.broadcasted_iota(jnp.int32, zc.shape, 1)
    cnt = jnp.sum(jnp.where(lane == 3, zc, 0.0), axis=1, keepdims=True)
    inv = 1.0 / jnp.maximum(cnt, 1.0)
    hbo_ref[:, :HALF] = hb_ref[0] * inv
    hbo_ref[:, HALF:] = hb_ref[1] * inv
    zbo_ref[...] = jnp.where(lane == 3, 0.0, zc * inv)


def _tc_mean(hbsum, zc):
    return pl.pallas_call(
        _mean_body,
        grid=(N_BLOCKS // NT,),
        in_specs=[pl.BlockSpec((2, NT, HALF), lambda t: (0, t, 0)),
                  pl.BlockSpec((NT, LANES), lambda t: (t, 0))],
        out_specs=[pl.BlockSpec((NT, HIDDEN), lambda t: (t, 0)),
                   pl.BlockSpec((NT, LANES), lambda t: (t, 0))],
        out_shape=[jax.ShapeDtypeStruct((N_BLOCKS, HIDDEN), jnp.float32),
                   jax.ShapeDtypeStruct((N_BLOCKS, LANES), jnp.float32)],
    )(hbsum, zc)


SCT = 1024


def _scale_body(dv_ref, dist_ref, sc_ref):
    t = pl.program_id(0)
    dv = dv_ref[...]
    d2 = jnp.sum(dv * dv, axis=1, keepdims=True).reshape(SCT // 128, 128)
    dist = jnp.sqrt(d2 + 1e-12)
    x = dist * (np.pi / CUTOFF)
    t2 = x * x
    cosx = 1.0 + t2 * (-1.0 / 2 + t2 * (1.0 / 24 + t2 * (-1.0 / 720 + t2 * (
        1.0 / 40320 + t2 * (-1.0 / 3628800 + t2 * (
            1.0 / 479001600 - t2 * (1.0 / 87178291200)))))))
    cut = 0.5 * (cosx + 1.0)
    cut = cut * (dist < CUTOFF).astype(jnp.float32)
    e = (lax.broadcasted_iota(jnp.int32, (SCT // 128, 128), 0) * 128
         + lax.broadcasted_iota(jnp.int32, (SCT // 128, 128), 1) + t * SCT)
    sc_ref[...] = cut * (e < N_EDGES).astype(jnp.float32)
    dist_ref[...] = dist


def _tc_scale(dvec):
    out_shape = jax.ShapeDtypeStruct((E_PAD // 128, 128), jnp.float32)
    out_spec = pl.BlockSpec((SCT // 128, 128), lambda t: (t, 0))
    return pl.pallas_call(
        _scale_body,
        grid=(E_PAD // SCT,),
        in_specs=[pl.BlockSpec((SCT, LANES), lambda t: (t, 0))],
        out_specs=[out_spec, out_spec],
        out_shape=[out_shape, out_shape],
    )(dvec)


def _wf_body(dist_ref, sc_ref, ea_ref, ew_ref, eb_ref,
             w1_ref, b1_ref, w2_ref, b2_ref, o_ref):
    dist = dist_ref[...]
    scale = sc_ref[...]
    spacing = CUTOFF / (NUM_GAUSSIANS - 1)
    coeff = -0.5 / spacing ** 2
    offs = lax.broadcasted_iota(
        jnp.int32, (TE, NUM_GAUSSIANS), 1).astype(jnp.float32) * spacing
    ef = jnp.exp(coeff * (dist - offs) ** 2)
    eap = jnp.dot(ea_ref[...], ew_ref[...],
                  preferred_element_type=jnp.float32) + eb_ref[...]
    ef = ef + eap
    s = jnp.dot(ef, w1_ref[...], preferred_element_type=jnp.float32) + b1_ref[...]
    s = jax.nn.softplus(s) - LOG2
    wf = jnp.dot(s, w2_ref[...], preferred_element_type=jnp.float32) + b2_ref[...]
    wf = wf * scale
    o_ref[0] = wf[:, :HALF]
    o_ref[1] = wf[:, HALF:]


def _tc_wf(dist_e, sc_e, ea, ew, eb, w1, b1, w2, b2):
    full = lambda *shape: pl.BlockSpec(shape, lambda t: (0,) * len(shape))
    return pl.pallas_call(
        _wf_body,
        grid=(E_PAD // TE,),
        in_specs=[pl.BlockSpec((TE, 1), lambda t: (t, 0)),
                  pl.BlockSpec((TE, 1), lambda t: (t, 0)),
                  pl.BlockSpec((TE, EDGE_SIZE), lambda t: (t, 0)),
                  full(EDGE_SIZE, NUM_GAUSSIANS),
                  full(1, NUM_GAUSSIANS),
                  full(NUM_GAUSSIANS, HIDDEN),
                  full(1, HIDDEN),
                  full(HIDDEN, HIDDEN),
                  full(1, HIDDEN)],
        out_specs=pl.BlockSpec((2, TE, HALF), lambda t: (0, t, 0)),
        out_shape=jax.ShapeDtypeStruct((2, E_PAD, HALF), jnp.float32),
    )(dist_e, sc_e, ea, ew, eb, w1, b1, w2, b2)


def _hw_body(h_ref, w_ref, o_ref):
    r = jnp.dot(h_ref[...], w_ref[...], preferred_element_type=jnp.float32)
    o_ref[0] = r[:, :HALF]
    o_ref[1] = r[:, HALF:]


def _tc_hw(h, w):
    return pl.pallas_call(
        _hw_body,
        grid=(N_BLOCKS // NT,),
        in_specs=[pl.BlockSpec((NT, HIDDEN), lambda t: (t, 0)),
                  pl.BlockSpec((HIDDEN, HIDDEN), lambda t: (0, 0))],
        out_specs=pl.BlockSpec((2, NT, HALF), lambda t: (0, t, 0)),
        out_shape=jax.ShapeDtypeStruct((2, N_BLOCKS, HALF), jnp.float32),
    )(h, w)


def _post_body(a_ref, h_ref, w2_ref, b2_ref, ow_ref, ob_ref, o_ref):
    x = (jnp.dot(a_ref[0], w2_ref[:HALF, :], preferred_element_type=jnp.float32)
         + jnp.dot(a_ref[1], w2_ref[HALF:, :], preferred_element_type=jnp.float32)
         + b2_ref[...])
    x = jax.nn.softplus(x) - LOG2
    x = jnp.dot(x, ow_ref[...], preferred_element_type=jnp.float32) + ob_ref[...]
    o_ref[...] = h_ref[...] + x


def _tc_post(agg, h, w2, b2, ow, ob):
    return pl.pallas_call(
        _post_body,
        grid=(N_BLOCKS // NT,),
        in_specs=[pl.BlockSpec((2, NT, HALF), lambda t: (0, t, 0)),
                  pl.BlockSpec((NT, HIDDEN), lambda t: (t, 0)),
                  pl.BlockSpec((HIDDEN, HIDDEN), lambda t: (0, 0)),
                  pl.BlockSpec((1, HIDDEN), lambda t: (0, 0)),
                  pl.BlockSpec((HIDDEN, HIDDEN), lambda t: (0, 0)),
                  pl.BlockSpec((1, HIDDEN), lambda t: (0, 0))],
        out_specs=pl.BlockSpec((NT, HIDDEN), lambda t: (t, 0)),
        out_shape=jax.ShapeDtypeStruct((N_BLOCKS, HIDDEN), jnp.float32),
    )(agg, h, w2, b2, ow, ob)


def _norm_body(x_ref, o_ref):
    x = x_ref[...]
    n = jnp.sqrt(jnp.sum(x * x, axis=1, keepdims=True))
    o_ref[...] = x / jnp.maximum(n, 1e-12)


def _tc_norm(x, tile):
    n = x.shape[0]
    return pl.pallas_call(
        _norm_body,
        grid=(n // tile,),
        in_specs=[pl.BlockSpec((tile, HIDDEN), lambda t: (t, 0))],
        out_specs=pl.BlockSpec((tile, HIDDEN), lambda t: (t, 0)),
        out_shape=jax.ShapeDtypeStruct((n, HIDDEN), jnp.float32),
    )(x)


def _pool_body(bid_ref, br_ref, o_ref):
    t = pl.program_id(0)

    @pl.when(t == 0)
    def _():
        o_ref[...] = jnp.zeros_like(o_ref)

    bid = bid_ref[0, 0, :]
    gi = lax.broadcasted_iota(jnp.int32, (N_GRAPHS, NT), 0)
    oh = (gi == bid[None, :]).astype(jnp.float32)
    o_ref[...] += jnp.dot(oh, br_ref[...], preferred_element_type=jnp.float32)


def _tc_pool(bid3, br):
    return pl.pallas_call(
        _pool_body,
        grid=(N_BLOCKS // NT,),
        in_specs=[pl.BlockSpec((1, 1, NT), lambda t: (t, 0, 0)),
                  pl.BlockSpec((NT, HIDDEN), lambda t: (t, 0))],
        out_specs=pl.BlockSpec((N_GRAPHS, HIDDEN), lambda t: (0, 0)),
        out_shape=jax.ShapeDtypeStruct((N_GRAPHS, HIDDEN), jnp.float32),
    )(bid3, br)


def kernel(H, Z, block_id, batch_id, edges, edge_attr, params):
    f32 = jnp.float32
    H = H.astype(f32)
    Z = Z.astype(f32)
    edge_attr = edge_attr.astype(f32)
    bid_p = jnp.pad(block_id.astype(jnp.int32), (0, A_PAD - N_ATOMS))
    rowp = jnp.pad(edges[0].astype(jnp.int32), (0, E_PAD - N_EDGES))
    colp = jnp.pad(edges[1].astype(jnp.int32), (0, E_PAD - N_EDGES))

    h_pad = jnp.pad(H, ((0, A_PAD - N_ATOMS), (0, 0)))
    zlane = jnp.concatenate(
        [Z, jnp.ones((N_ATOMS, 1), f32), jnp.zeros((N_ATOMS, 12), f32)], axis=1)
    zp = jnp.pad(zlane, ((0, A_PAD - N_ATOMS), (0, 0)))
    ea_p = jnp.pad(edge_attr, ((0, E_PAD - N_EDGES), (0, 0)))

    hsplit = _tc_split(h_pad).reshape(2 * A_PAD, HALF)
    hbsum2, zc2 = _sc_atoms_call(hsplit, zp, bid_p)
    hb, zb16 = _tc_mean(hbsum2.reshape(2, N_BLOCKS, HALF), zc2[:N_BLOCKS])

    dvec = _sc_dvec_call(zb16, rowp, colp)

    lp = params['layers']
    dist_e, sc_e = _tc_scale(dvec)
    dist_c = dist_e.reshape(E_PAD, 1)
    sc_c = sc_e.reshape(E_PAD, 1)
    ew = params['edge_linear_w'].astype(f32)
    eb = params['edge_linear_b'].astype(f32).reshape(1, NUM_GAUSSIANS)
    row2 = rowp.reshape(E_PAD // C32, C32)
    col2 = colp.reshape(E_PAD // C32, C32)

    def wf_layer(l):
        return _tc_wf(dist_c, sc_c, ea_p, ew, eb,
                      lp[l]['mlp_w1'].astype(f32),
                      lp[l]['mlp_b1'].astype(f32).reshape(1, HIDDEN),
                      lp[l]['mlp_w2'].astype(f32),
                      lp[l]['mlp_b2'].astype(f32).reshape(1, HIDDEN))

    h = hb
    wf = wf_layer(0)
    for l in range(N_LAYERS):
        hw2 = _tc_hw(h, lp[l]['conv_lin1_w'].astype(f32))
        agg2 = _sc_edge_call(hw2.reshape(2 * N_BLOCKS, HALF),
                             wf.reshape(2 * E_PAD, HALF), row2, col2)
        if l + 1 < N_LAYERS:
            wf = wf_layer(l + 1)
        h = _tc_post(agg2.reshape(2, N_BLOCKS, HALF), h,
                     lp[l]['conv_lin2_w'].astype(f32),
                     lp[l]['conv_lin2_b'].astype(f32).reshape(1, HIDDEN),
                     lp[l]['out_w'].astype(f32),
                     lp[l]['out_b'].astype(f32).reshape(1, HIDDEN))

    block_repr = _tc_norm(h, NT)
    graph = _tc_pool(batch_id.astype(jnp.int32).reshape(N_BLOCKS // NT, 1, NT),
                     block_repr)
    graph_repr = _tc_norm(graph, N_GRAPHS)
    return (hb, block_repr, graph_repr, None)
